# baseline (speedup 1.0000x reference)
import jax
import jax.numpy as jnp
from jax.experimental import pallas as pl
from jax.experimental.pallas import tpu as pltpu

_EXP_MIN = -10
_DIM_EXP = 64
_DIM_MAN = 64
_NUM_EXP = 31
_LN10 = 2.302585092994046
_ROWS = 8
_CH = 1024


def _tile_kernel(num_ref, rhs_ref, out_ref):
    x = num_ref[...]
    c = jnp.clip(x, 1e-10, 1e20)
    e = jnp.floor(jnp.log10(c + 1e-10))
    m = c / jnp.exp(e * _LN10)
    idx = (e - _EXP_MIN).astype(jnp.int32)
    msk = (x != 0).astype(jnp.float32)

    sub32 = jax.lax.broadcasted_iota(jnp.int32, (32, _CH), 0)
    proto = jax.lax.broadcasted_iota(
        jnp.int32, (_DIM_MAN, _CH), 0).astype(jnp.float32) * (20.0 / (_DIM_MAN - 1)) - 10.0
    rhs = rhs_ref[...]

    for r in range(_ROWS):
        idx_r = jnp.broadcast_to(idx[r:r + 1, :], (32, _CH))
        m_r = jnp.broadcast_to(m[r:r + 1, :], (_DIM_MAN, _CH))
        msk_r = jnp.broadcast_to(msk[r:r + 1, :], (96, _CH))
        onehot_t = (sub32 == idx_r).astype(jnp.float32)
        d = m_r - proto
        man_t = jnp.exp(-d * d)
        lhs_t = (jnp.concatenate([onehot_t, man_t], axis=0) * msk_r
                 ).astype(jnp.bfloat16)
        chunk = jax.lax.dot_general(
            lhs_t, rhs,
            dimension_numbers=(((0,), (0,)), ((), ())),
            preferred_element_type=jnp.float32,
        )
        out_ref[r * _CH:(r + 1) * _CH, :] = chunk


def kernel(numbers, exp_table):
    rows, cols = numbers.shape
    n = rows * cols
    blk = _ROWS * _CH
    grid = n // blk
    nums2d = numbers.reshape(n // _CH, _CH)
    tab = jnp.pad(exp_table, ((0, 32 - _NUM_EXP), (0, 0)))
    top = jnp.concatenate([tab, jnp.zeros((32, _DIM_MAN), jnp.float32)], axis=1)
    bot = jnp.concatenate(
        [jnp.zeros((_DIM_MAN, _DIM_EXP), jnp.float32),
         jnp.eye(_DIM_MAN, dtype=jnp.float32)], axis=1)
    rhs = jnp.concatenate([top, bot], axis=0).astype(jnp.bfloat16)
    out = pl.pallas_call(
        _tile_kernel,
        grid=(grid,),
        in_specs=[
            pl.BlockSpec((_ROWS, _CH), lambda i: (i, 0)),
            pl.BlockSpec((96, 128), lambda i: (0, 0)),
        ],
        out_specs=pl.BlockSpec((blk, _DIM_EXP + _DIM_MAN), lambda i: (i, 0)),
        out_shape=jax.ShapeDtypeStruct((n, _DIM_EXP + _DIM_MAN), jnp.float32),
        compiler_params=pltpu.CompilerParams(
            dimension_semantics=("arbitrary",),
        ),
    )(nums2d, rhs)
    return out.reshape(rows, cols, _DIM_EXP + _DIM_MAN)


# speedup vs baseline: 1.1332x; 1.1332x over previous
import jax
import jax.numpy as jnp
from jax.experimental import pallas as pl
from jax.experimental.pallas import tpu as pltpu

_EXP_MIN = -10
_DIM_EXP = 64
_DIM_MAN = 64
_NUM_EXP = 31
_LN10 = 2.302585092994046
_ROWS = 8
_CH = 2048


def _tile_kernel(num_ref, rhs_ref, out_ref):
    x = num_ref[...]
    c = jnp.clip(x, 1e-10, 1e20)
    e = jnp.floor(jnp.log10(c + 1e-10))
    m = c / jnp.exp(e * _LN10)
    idx = (e - _EXP_MIN).astype(jnp.int32)
    msk = (x != 0).astype(jnp.float32)

    sub32 = jax.lax.broadcasted_iota(jnp.int32, (32, _CH), 0)
    proto = jax.lax.broadcasted_iota(
        jnp.int32, (_DIM_MAN, _CH), 0).astype(jnp.float32) * (20.0 / (_DIM_MAN - 1)) - 10.0
    rhs = rhs_ref[...]

    for r in range(_ROWS):
        idx_r = jnp.broadcast_to(idx[r:r + 1, :], (32, _CH))
        m_r = jnp.broadcast_to(m[r:r + 1, :], (_DIM_MAN, _CH))
        msk_r = jnp.broadcast_to(msk[r:r + 1, :], (96, _CH))
        onehot_t = (sub32 == idx_r).astype(jnp.float32)
        d = m_r - proto
        man_t = jnp.exp(-d * d)
        lhs_t = (jnp.concatenate([onehot_t, man_t], axis=0) * msk_r
                 ).astype(jnp.bfloat16)
        chunk = jax.lax.dot_general(
            lhs_t, rhs,
            dimension_numbers=(((0,), (0,)), ((), ())),
            preferred_element_type=jnp.float32,
        )
        out_ref[r * _CH:(r + 1) * _CH, :] = chunk


def kernel(numbers, exp_table):
    rows, cols = numbers.shape
    n = rows * cols
    blk = _ROWS * _CH
    grid = n // blk
    nums2d = numbers.reshape(n // _CH, _CH)
    tab = jnp.pad(exp_table, ((0, 32 - _NUM_EXP), (0, 0)))
    top = jnp.concatenate([tab, jnp.zeros((32, _DIM_MAN), jnp.float32)], axis=1)
    bot = jnp.concatenate(
        [jnp.zeros((_DIM_MAN, _DIM_EXP), jnp.float32),
         jnp.eye(_DIM_MAN, dtype=jnp.float32)], axis=1)
    rhs = jnp.concatenate([top, bot], axis=0).astype(jnp.bfloat16)
    out = pl.pallas_call(
        _tile_kernel,
        grid=(grid,),
        in_specs=[
            pl.BlockSpec((_ROWS, _CH), lambda i: (i, 0)),
            pl.BlockSpec((96, 128), lambda i: (0, 0)),
        ],
        out_specs=pl.BlockSpec((blk, _DIM_EXP + _DIM_MAN), lambda i: (i, 0)),
        out_shape=jax.ShapeDtypeStruct((n, _DIM_EXP + _DIM_MAN), jnp.float32),
        compiler_params=pltpu.CompilerParams(
            dimension_semantics=("arbitrary",),
        ),
    )(nums2d, rhs)
    return out.reshape(rows, cols, _DIM_EXP + _DIM_MAN)
